# Initial kernel scaffold; baseline (speedup 1.0000x reference)
import jax
import jax.numpy as jnp
from jax.experimental import pallas as pl


def kernel(x, edge_index, edge_attr, Wrel0, Wroot0, b0, Wrel1, Wroot1, b1, Wrel2, Wroot2, b2, fc1W, fc1b, fc2W, fc2b):
    raise NotImplementedError("write your pallas kernel here")


# speedup vs baseline: 3.1062x; 3.1062x over previous
import functools

import jax
import jax.numpy as jnp
from jax import lax
from jax.experimental import pallas as pl
from jax.experimental.pallas import tpu as pltpu, tpu_sc as plsc

N = 10000
E = 320000
H = 128
R = 4
NCLASS = 8

NTILES = 16
NW = 32
EPW = E // NW
ECH = 2000
CB = 128
NSEG = 5
SEGN = 2400
SEG_ROWS = 2560
TRASH = SEGN
NB = R * NSEG
PADN = NSEG * SEG_ROWS
LISTR = 112
MROWS = 96
BN = 80
GRID_N = N // BN

_SC_PARAMS = pltpu.CompilerParams(needs_layout_passes=False)


def _agg_block_index(i):
    return (i // 30) * 32 + (i % 30)


def _sc_compact_body(bkt2, gi2, ld2, cg3, cs3, npad_out,
                     bbuf, gbuf, lbuf, lcg, lcs, hist, ptrtbl, npads8):
    c = lax.axis_index("c")
    s = lax.axis_index("s")
    t = c * NTILES + s
    zero16 = jnp.zeros((16,), jnp.int32)
    hist[pl.ds(0, 16)] = zero16
    hist[pl.ds(16, 16)] = zero16

    for cc in range(EPW // ECH):
        off = pl.multiple_of(t * EPW + cc * ECH, 8)
        pltpu.sync_copy(bkt2.at[pl.ds(off, ECH)], bbuf)

        def count_body(i, carry):
            b = bbuf[pl.ds(i * 16, 16)]
            cnts, last = plsc.scan_count(b)
            plsc.addupdate_scatter(hist, [b], cnts, mask=last)
            return carry

        lax.fori_loop(0, ECH // 16, count_body, 0, unroll=2)

    ha = hist[pl.ds(0, 16)]
    hb = hist[pl.ds(16, 16)]
    npv_a = ((ha + (CB - 1)) // CB) * CB
    npv_b = ((hb + (CB - 1)) // CB) * CB
    cums_a = plsc.cumsum(npv_a)
    loff_a = cums_a - npv_a
    tot_a = jnp.max(cums_a)
    cums_b = plsc.cumsum(npv_b)
    loff_b = cums_b - npv_b + tot_a
    ptrtbl[pl.ds(0, 16)] = loff_a
    ptrtbl[pl.ds(16, 16)] = loff_b

    def fill_body(i, carry):
        for k in range(8):
            lcg[i, pl.ds(k * 16, 16)] = zero16
            lcs[i, pl.ds(k * 16, 16)] = zero16 + TRASH
        return carry

    lax.fori_loop(0, LISTR, fill_body, 0, unroll=2)

    for cc in range(EPW // ECH):
        off = pl.multiple_of(t * EPW + cc * ECH, 8)
        pltpu.sync_copy(bkt2.at[pl.ds(off, ECH)], bbuf)
        pltpu.sync_copy(gi2.at[pl.ds(off, ECH)], gbuf)
        pltpu.sync_copy(ld2.at[pl.ds(off, ECH)], lbuf)

        def place_body(i, carry):
            b = bbuf[pl.ds(i * 16, 16)]
            g = gbuf[pl.ds(i * 16, 16)]
            l = lbuf[pl.ds(i * 16, 16)]
            cnts, last = plsc.scan_count(b)
            base = plsc.load_gather(ptrtbl, [b])
            dest = base + cnts - 1
            dr = dest // CB
            dc = dest - dr * CB
            plsc.store_scatter(lcg, [dr, dc], g)
            plsc.store_scatter(lcs, [dr, dc], l)
            plsc.addupdate_scatter(ptrtbl, [b], cnts, mask=last)
            return carry

        lax.fori_loop(0, ECH // 16, place_body, 0, unroll=2)

    npads8[0, pl.ds(0, 16)] = npv_a
    npads8[0, pl.ds(16, 16)] = npv_b
    for k in range(1, 8):
        npads8[k, pl.ds(0, 16)] = zero16
        npads8[k, pl.ds(16, 16)] = zero16
    pltpu.sync_copy(lcg, cg3.at[t])
    pltpu.sync_copy(lcs, cs3.at[t])
    pltpu.sync_copy(npads8, npad_out.at[t])


_sc_compact = pl.kernel(
    _sc_compact_body,
    out_type=(jax.ShapeDtypeStruct((NW, LISTR, CB), jnp.int32),
              jax.ShapeDtypeStruct((NW, LISTR, CB), jnp.int32),
              jax.ShapeDtypeStruct((NW, 8, 32), jnp.int32)),
    mesh=plsc.VectorSubcoreMesh(core_axis_name="c", subcore_axis_name="s"),
    scratch_types=[
        pltpu.VMEM((ECH,), jnp.int32),
        pltpu.VMEM((ECH,), jnp.int32),
        pltpu.VMEM((ECH,), jnp.int32),
        pltpu.VMEM((LISTR, CB), jnp.int32),
        pltpu.VMEM((LISTR, CB), jnp.int32),
        pltpu.VMEM((32,), jnp.int32),
        pltpu.VMEM((32,), jnp.int32),
        pltpu.VMEM((8, 32), jnp.int32),
    ],
    compiler_params=_SC_PARAMS,
)


def _sc_consume_body(with_counts, *refs):
    if with_counts:
        (cs3, npad_h, ones_h, zb_h,
         cnt_out,
         ms, npbuf, ones_v, zb, agg, gs0) = refs
        ht = cg3 = mg = rows = None
    else:
        (ht, cg3, cs3, npad_h, zb_h,
         agg_out,
         mg, ms, rows, npbuf, zb, agg, gs0) = refs
        ones_v = None
    c = lax.axis_index("c")
    s = lax.axis_index("s")
    lane = lax.iota(jnp.int32, 16)
    pltpu.sync_copy(zb_h, zb)
    for sti in range(2):
        pltpu.sync_copy(npad_h.at[2 * s + sti], npbuf.at[sti])
    if with_counts:
        pltpu.sync_copy(ones_h, ones_v)
    for j in range(3):
        seg = c + 2 * j
        for r in range(R):
            q = r * NSEG + seg

            @pl.when(seg < NSEG)
            def _pass():
                for k in range(5):
                    pltpu.async_copy(zb, agg.at[pl.ds(s * 160 + k * 32, 32)],
                                     gs0)
                for k in range(5):
                    pltpu.make_async_copy(
                        zb, agg.at[pl.ds(s * 160 + k * 32, 32)], gs0).wait()
                plsc.subcore_barrier()
                for sti in range(2):
                    st = 2 * s + sti
                    np_a = npbuf[sti, 0, pl.ds(0, 16)]
                    np_b = npbuf[sti, 0, pl.ds(16, 16)]
                    rw_a = np_a // CB
                    rw_b = np_b // CB
                    nch = (jnp.sum(jnp.where(lane == q, rw_a, 0))
                           + jnp.sum(jnp.where(lane + 16 == q, rw_b, 0)))
                    loff = (jnp.sum(jnp.where(lane < q, rw_a, 0))
                            + jnp.sum(jnp.where(lane + 16 < q, rw_b, 0)))
                    start8 = (loff // 8) * 8
                    skip = loff - start8

                    def mload(bi, carry):
                        @pl.when(bi * 8 < skip + nch)
                        def _():
                            if not with_counts:
                                pltpu.sync_copy(
                                    cg3.at[st, pl.ds(start8 + bi * 8, 8)],
                                    mg.at[pl.ds(bi * 8, 8)])
                            pltpu.sync_copy(
                                cs3.at[st, pl.ds(start8 + bi * 8, 8)],
                                ms.at[pl.ds(bi * 8, 8)])
                        return carry

                    lax.fori_loop(0, MROWS // 8, mload, 0)

                    def chunk(k, carry):
                        @pl.when(k < nch)
                        def _():
                            if with_counts:
                                pltpu.sync_copy(ones_v,
                                                agg.at[ms.at[skip + k]],
                                                add=True)
                            else:
                                pltpu.async_copy(ht.at[mg.at[skip + k]],
                                                 rows, gs0).wait()
                                pltpu.sync_copy(rows,
                                                agg.at[ms.at[skip + k]],
                                                add=True)
                        return carry

                    lax.fori_loop(0, MROWS - 8, chunk, 0)
                plsc.subcore_barrier()
                out_ref = cnt_out if with_counts else agg_out
                pltpu.sync_copy(
                    agg.at[pl.ds(s * 160, 160)],
                    out_ref.at[r, pl.ds(seg * SEG_ROWS + s * 160, 160)])
                plsc.subcore_barrier()


def _make_sc_consume(with_counts):
    out_type = [jax.ShapeDtypeStruct((R, PADN, H), jnp.float32)]
    scratch = []
    if not with_counts:
        scratch.append(pltpu.VMEM((MROWS, CB), jnp.int32))
    scratch.append(pltpu.VMEM((MROWS, CB), jnp.int32))
    if not with_counts:
        scratch.append(pltpu.VMEM((CB, H), jnp.float32))
    scratch.append(pltpu.VMEM((2, 8, 32), jnp.int32))
    if with_counts:
        scratch.append(pltpu.VMEM((CB, H), jnp.float32))
    scratch.append(pltpu.VMEM((32, H), jnp.float32))
    scratch.append(pltpu.VMEM_SHARED((SEG_ROWS, H), jnp.float32))
    scratch.append(pltpu.SemaphoreType.DMA)
    return pl.kernel(
        functools.partial(_sc_consume_body, with_counts),
        out_type=tuple(out_type),
        mesh=plsc.VectorSubcoreMesh(core_axis_name="c", subcore_axis_name="s"),
        scratch_types=scratch,
        compiler_params=_SC_PARAMS,
    )


_sc_count = _make_sc_consume(True)
_sc_consume_plain = _make_sc_consume(False)


def _rel_mm_body(h_ref, w_ref, o_ref):
    o_ref[0] = jnp.dot(h_ref[...], w_ref[0],
                       preferred_element_type=jnp.float32)


_rel_mm = pl.pallas_call(
    _rel_mm_body,
    grid=(R, 25),
    in_specs=[
        pl.BlockSpec((400, H), lambda r, i: (i, 0)),
        pl.BlockSpec((1, H, H), lambda r, i: (r, 0, 0)),
    ],
    out_specs=pl.BlockSpec((1, 400, H), lambda r, i: (r, i, 0)),
    out_shape=jax.ShapeDtypeStruct((R, N, H), jnp.float32),
)


def _combine_body(h_ref, wroot_ref, b_ref, agg_ref, cnt_ref, o_ref):
    acc = jnp.dot(h_ref[...], wroot_ref[...],
                  preferred_element_type=jnp.float32) + b_ref[0]
    inv = 1.0 / jnp.maximum(cnt_ref[...], 1.0)
    for r in range(R):
        acc = acc + agg_ref[r] * inv[:, r:r + 1]
    o_ref[...] = jnp.maximum(acc, 0.0)


_combine = pl.pallas_call(
    _combine_body,
    grid=(GRID_N,),
    in_specs=[
        pl.BlockSpec((BN, H), lambda i: (i, 0)),
        pl.BlockSpec((H, H), lambda i: (0, 0)),
        pl.BlockSpec((1, H), lambda i: (0, 0)),
        pl.BlockSpec((R, BN, H), lambda i: (0, _agg_block_index(i), 0)),
        pl.BlockSpec((BN, R), lambda i: (i, 0)),
    ],
    out_specs=pl.BlockSpec((BN, H), lambda i: (i, 0)),
    out_shape=jax.ShapeDtypeStruct((N, H), jnp.float32),
)


def _head_body(h1_ref, h2_ref, h3_ref, w1_ref, b1_ref, w2_ref, b2_ref, o_ref):
    z = jnp.concatenate([h1_ref[...], h2_ref[...], h3_ref[...]], axis=1)
    z = jnp.dot(z, w1_ref[...], preferred_element_type=jnp.float32) + b1_ref[0]
    z = jnp.maximum(z, 0.0)
    logits = jnp.dot(z, w2_ref[...],
                     preferred_element_type=jnp.float32) + b2_ref[0]
    m = jnp.max(logits, axis=1, keepdims=True)
    lse = jnp.log(jnp.sum(jnp.exp(logits - m), axis=1, keepdims=True)) + m
    o_ref[...] = logits - lse


_head = pl.pallas_call(
    _head_body,
    grid=(25,),
    in_specs=[
        pl.BlockSpec((400, H), lambda i: (i, 0)),
        pl.BlockSpec((400, H), lambda i: (i, 0)),
        pl.BlockSpec((400, H), lambda i: (i, 0)),
        pl.BlockSpec((3 * H, H), lambda i: (0, 0)),
        pl.BlockSpec((1, H), lambda i: (0, 0)),
        pl.BlockSpec((H, NCLASS), lambda i: (0, 0)),
        pl.BlockSpec((1, NCLASS), lambda i: (0, 0)),
    ],
    out_specs=pl.BlockSpec((400, NCLASS), lambda i: (i, 0)),
    out_shape=jax.ShapeDtypeStruct((N, NCLASS), jnp.float32),
)


def kernel(x, edge_index, edge_attr,
           Wrel0, Wroot0, b0,
           Wrel1, Wroot1, b1,
           Wrel2, Wroot2, b2,
           fc1W, fc1b, fc2W, fc2b):
    src = edge_index[0]
    dst = edge_index[1]
    et = edge_attr.astype(jnp.int32)
    seg_of = dst // SEGN
    bkt2 = et * NSEG + seg_of
    gi2 = et * N + src
    ld2 = dst - seg_of * SEGN
    zb = jnp.zeros((32, H), jnp.float32)
    ones_v = jnp.ones((CB, H), jnp.float32)

    cg3, cs3, npad = _sc_compact(bkt2, gi2, ld2)
    (cnt_w,) = _sc_count(cs3, npad, ones_v, zb)
    cnt = (cnt_w[:, :, 0]
           .reshape(R, NSEG, SEG_ROWS)[:, :, :SEGN]
           .reshape(R, NSEG * SEGN)[:, :N].T)

    Wrels = [Wrel0, Wrel1, Wrel2]
    Wroots = [Wroot0, Wroot1, Wroot2]
    bs = [b0, b1, b2]

    h = x
    hs = []
    for i in range(3):
        ht = _rel_mm(h, Wrels[i]).reshape(R * N, H)
        (agg,) = _sc_consume_plain(ht, cg3, cs3, npad, zb)
        h = _combine(h, Wroots[i], bs[i].reshape(1, H), agg, cnt)
        hs.append(h)
    return _head(hs[0], hs[1], hs[2],
                 fc1W, fc1b.reshape(1, H), fc2W, fc2b.reshape(1, NCLASS))
